# baseline (speedup 1.0000x reference)
import functools

import jax
import jax.numpy as jnp
import numpy as np
from jax.experimental import pallas as pl
from jax.experimental.pallas import tpu as pltpu

_B, _N, _C, _H = 2, 1024, 256, 8
_HD = _C // _H
_NSUB = 128
_BQ = 128
_SCALE = _HD ** (-0.5)


def _qkv_kernel(x_ref, w_ref, o_ref):
    x = x_ref[0]
    w = w_ref[...]
    o_ref[0] = jax.lax.dot_general(
        x, w, (((1,), (1,)), ((), ())), preferred_element_type=jnp.float32
    )


def _attn_kernel(q_ref, k_ref, v_ref, relt_ref, ccol_ref, crow_ref,
                 wrel_ref, wproj_ref, bproj_ref, brel_ref, o_ref):
    ccol = ccol_ref[0]
    crow = crow_ref[0]

    d2 = jnp.zeros((_BQ, _N), dtype=jnp.float32)
    for a in range(3):
        diff = ccol[:, a:a + 1] - crow[a:a + 1, :]
        d2 = d2 + diff * diff
    dist = jnp.sqrt(d2)
    bits = jax.lax.bitcast_convert_type(dist, jnp.int32)

    lo0 = jnp.full((_BQ, 1), -1, dtype=jnp.int32)
    hi0 = jnp.full((_BQ, 1), 0x7F800000, dtype=jnp.int32)

    def bs_body(_, carry):
        lo, hi = carry
        mid = lo + ((hi - lo) >> 1)
        cnt = jnp.sum((bits <= mid).astype(jnp.int32), axis=1, keepdims=True)
        ge = cnt >= _NSUB
        return jnp.where(ge, lo, mid), jnp.where(ge, mid, hi)

    _, tbits = jax.lax.fori_loop(0, 31, bs_body, (lo0, hi0))

    less = bits < tbits
    tie = bits == tbits
    need = _NSUB - jnp.sum(less.astype(jnp.int32), axis=1, keepdims=True)

    jidx = jax.lax.broadcasted_iota(jnp.int32, (_BQ, _N), 1)
    lo1 = jnp.full((_BQ, 1), -1, dtype=jnp.int32)
    hi1 = jnp.full((_BQ, 1), _N - 1, dtype=jnp.int32)

    def is_body(_, carry):
        lo, hi = carry
        mid = lo + ((hi - lo) >> 1)
        cnt = jnp.sum((tie & (jidx <= mid)).astype(jnp.int32), axis=1,
                      keepdims=True)
        ge = cnt >= need
        return jnp.where(ge, lo, mid), jnp.where(ge, mid, hi)

    _, u = jax.lax.fori_loop(0, 11, is_body, (lo1, hi1))
    sel = less | (tie & (jidx <= u))

    q = q_ref[0]
    k = k_ref[0]
    v = v_ref[0]
    relt = relt_ref[0]
    wrel = wrel_ref[...]

    outs = [None] * _H
    for hp in range(_H // 2):
        ha, hb = 2 * hp, 2 * hp + 1
        sla = slice(ha * _HD, (ha + 1) * _HD)
        slb = slice(hb * _HD, (hb + 1) * _HD)
        qa, qb = q[:, sla], q[:, slb]
        ska = jax.lax.dot_general(qa, k[:, sla], (((1,), (1,)), ((), ())),
                                  preferred_element_type=jnp.float32)
        skb = jax.lax.dot_general(qb, k[:, slb], (((1,), (1,)), ((), ())),
                                  preferred_element_type=jnp.float32)
        sqa = jax.lax.dot_general(qa, wrel[sla, :], (((1,), (0,)), ((), ())),
                                  preferred_element_type=jnp.float32)
        sqb = jax.lax.dot_general(qb, wrel[slb, :], (((1,), (0,)), ((), ())),
                                  preferred_element_type=jnp.float32)
        ca, cb = [], []
        for jc in range(_N // 128):
            js = slice(jc * 128, (jc + 1) * 128)
            aa = ska[:, js]
            ab = skb[:, js]
            for r in range(12):
                c = relt[r][:, js]
                aa = aa + sqa[:, r:r + 1] * c
                ab = ab + sqb[:, r:r + 1] * c
            ca.append(aa)
            cb.append(ab)
        for h, chunks, sq_, slh in ((ha, ca, sqa, sla), (hb, cb, sqb, slb)):
            s = jnp.concatenate(chunks, axis=1)
            s = jnp.where(sel, s * _SCALE, -jnp.inf)
            m = jnp.max(s, axis=1, keepdims=True)
            e = jnp.exp(s - m)
            inv = 1.0 / jnp.sum(e, axis=1, keepdims=True)
            oh = jax.lax.dot_general(e, v[:, slh], (((1,), (0,)), ((), ())),
                                     preferred_element_type=jnp.float32)
            t = jnp.concatenate(
                [jnp.sum(e * relt[r], axis=1, keepdims=True)
                 for r in range(12)], axis=1)
            oh = oh + jax.lax.dot_general(t, wrel[slh, :],
                                          (((1,), (1,)), ((), ())),
                                          preferred_element_type=jnp.float32)
            oh = oh * inv
            oh = oh + brel_ref[:, slh]
            outs[h] = oh

    out = jnp.concatenate(outs, axis=1)
    o_ref[0] = jax.lax.dot_general(out, wproj_ref[...],
                                   (((1,), (1,)), ((), ())),
                                   preferred_element_type=jnp.float32
                                   ) + bproj_ref[...]


@jax.jit
def kernel(x, rel, Wqkv, Wproj, bproj, Wrel, brel):
    nblk = _N // _BQ

    qkv = pl.pallas_call(
        _qkv_kernel,
        grid=(_B, nblk),
        in_specs=[
            pl.BlockSpec((1, _BQ, _C), lambda b, i: (b, i, 0)),
            pl.BlockSpec((3 * _C, _C), lambda b, i: (0, 0)),
        ],
        out_specs=pl.BlockSpec((1, _BQ, 3 * _C), lambda b, i: (b, i, 0)),
        out_shape=jax.ShapeDtypeStruct((_B, _N, 3 * _C), jnp.float32),
        compiler_params=pltpu.CompilerParams(
            dimension_semantics=("parallel", "parallel")),
    )(x, Wqkv)
    q = qkv[:, :, :_C]
    k = qkv[:, :, _C:2 * _C]
    v = qkv[:, :, 2 * _C:]

    centers = rel[:, 0, :, 0:3]
    ccol = jnp.concatenate(
        [centers, jnp.zeros((_B, _N, 5), jnp.float32)], axis=-1)
    crow = jnp.transpose(ccol, (0, 2, 1))
    rel_t = jnp.transpose(rel, (0, 3, 1, 2))

    out = pl.pallas_call(
        _attn_kernel,
        grid=(_B, nblk),
        in_specs=[
            pl.BlockSpec((1, _BQ, _C), lambda b, i: (b, i, 0)),
            pl.BlockSpec((1, _N, _C), lambda b, i: (b, 0, 0)),
            pl.BlockSpec((1, _N, _C), lambda b, i: (b, 0, 0)),
            pl.BlockSpec((1, 12, _BQ, _N), lambda b, i: (b, 0, i, 0)),
            pl.BlockSpec((1, _BQ, 8), lambda b, i: (b, i, 0)),
            pl.BlockSpec((1, 8, _N), lambda b, i: (b, 0, 0)),
            pl.BlockSpec((_C, 12), lambda b, i: (0, 0)),
            pl.BlockSpec((_C, _C), lambda b, i: (0, 0)),
            pl.BlockSpec((1, _C), lambda b, i: (0, 0)),
            pl.BlockSpec((1, _C), lambda b, i: (0, 0)),
        ],
        out_specs=pl.BlockSpec((1, _BQ, _C), lambda b, i: (b, i, 0)),
        out_shape=jax.ShapeDtypeStruct((_B, _N, _C), jnp.float32),
        compiler_params=pltpu.CompilerParams(
            dimension_semantics=("parallel", "parallel")),
    )(q, k, v, rel_t, ccol, crow, Wrel, Wproj,
      bproj.reshape(1, _C), brel.reshape(1, _C))
    return out


# speedup vs baseline: 1.2995x; 1.2995x over previous
import functools

import jax
import jax.numpy as jnp
import numpy as np
from jax.experimental import pallas as pl
from jax.experimental.pallas import tpu as pltpu

_B, _N, _C, _H = 2, 1024, 256, 8
_HD = _C // _H
_NSUB = 128
_BQ = 128
_SCALE = _HD ** (-0.5)


def _qkv_kernel(x_ref, w_ref, o_ref):
    x = x_ref[0]
    w = w_ref[...]
    o_ref[0] = jax.lax.dot_general(
        x, w, (((1,), (1,)), ((), ())), preferred_element_type=jnp.float32
    )


def _attn_kernel(q_ref, k_ref, v_ref, relt_ref, ccol_ref, crow_ref,
                 wrel_ref, wproj_ref, bproj_ref, brel_ref, o_ref):
    ccol = ccol_ref[0]
    crow = crow_ref[0]

    d2 = jnp.zeros((_BQ, _N), dtype=jnp.float32)
    for a in range(3):
        diff = ccol[:, a:a + 1] - crow[a:a + 1, :]
        d2 = d2 + diff * diff
    dist = jnp.sqrt(d2)
    bits = jax.lax.bitcast_convert_type(dist, jnp.int32)

    lo0 = jnp.full((_BQ, 1), -1, dtype=jnp.int32)
    hi0 = jnp.full((_BQ, 1), 0x7F800000, dtype=jnp.int32)

    def bs_body(_, carry):
        lo, hi = carry
        mid = lo + ((hi - lo) >> 1)
        cnt = jnp.sum((bits <= mid).astype(jnp.int32), axis=1, keepdims=True)
        ge = cnt >= _NSUB
        return jnp.where(ge, lo, mid), jnp.where(ge, mid, hi)

    carry = (lo0, hi0)
    for _ in range(31):
        carry = bs_body(0, carry)
    _, tbits = carry

    less = bits < tbits
    tie = bits == tbits
    need = _NSUB - jnp.sum(less.astype(jnp.int32), axis=1, keepdims=True)

    jidx = jax.lax.broadcasted_iota(jnp.int32, (_BQ, _N), 1)
    lo1 = jnp.full((_BQ, 1), -1, dtype=jnp.int32)
    hi1 = jnp.full((_BQ, 1), _N - 1, dtype=jnp.int32)

    def is_body(_, carry):
        lo, hi = carry
        mid = lo + ((hi - lo) >> 1)
        cnt = jnp.sum((tie & (jidx <= mid)).astype(jnp.int32), axis=1,
                      keepdims=True)
        ge = cnt >= need
        return jnp.where(ge, lo, mid), jnp.where(ge, mid, hi)

    carry = (lo1, hi1)
    for _ in range(11):
        carry = is_body(0, carry)
    _, u = carry
    sel = less | (tie & (jidx <= u))

    q = q_ref[0]
    k = k_ref[0]
    v = v_ref[0]
    relt = relt_ref[0]
    wrel = wrel_ref[...]

    outs = [None] * _H
    for hp in range(_H // 2):
        ha, hb = 2 * hp, 2 * hp + 1
        sla = slice(ha * _HD, (ha + 1) * _HD)
        slb = slice(hb * _HD, (hb + 1) * _HD)
        qa, qb = q[:, sla], q[:, slb]
        ska = jax.lax.dot_general(qa, k[:, sla], (((1,), (1,)), ((), ())),
                                  preferred_element_type=jnp.float32)
        skb = jax.lax.dot_general(qb, k[:, slb], (((1,), (1,)), ((), ())),
                                  preferred_element_type=jnp.float32)
        sqa = jax.lax.dot_general(qa, wrel[sla, :], (((1,), (0,)), ((), ())),
                                  preferred_element_type=jnp.float32)
        sqb = jax.lax.dot_general(qb, wrel[slb, :], (((1,), (0,)), ((), ())),
                                  preferred_element_type=jnp.float32)
        ca, cb = [], []
        for jc in range(_N // 128):
            js = slice(jc * 128, (jc + 1) * 128)
            aa = ska[:, js]
            ab = skb[:, js]
            for r in range(12):
                c = relt[r][:, js]
                aa = aa + sqa[:, r:r + 1] * c
                ab = ab + sqb[:, r:r + 1] * c
            ca.append(aa)
            cb.append(ab)
        for h, chunks, sq_, slh in ((ha, ca, sqa, sla), (hb, cb, sqb, slb)):
            s = jnp.concatenate(chunks, axis=1)
            s = jnp.where(sel, s * _SCALE, -jnp.inf)
            m = jnp.max(s, axis=1, keepdims=True)
            e = jnp.exp(s - m)
            inv = 1.0 / jnp.sum(e, axis=1, keepdims=True)
            oh = jax.lax.dot_general(e, v[:, slh], (((1,), (0,)), ((), ())),
                                     preferred_element_type=jnp.float32)
            t = jnp.concatenate(
                [jnp.sum(e * relt[r], axis=1, keepdims=True)
                 for r in range(12)], axis=1)
            oh = oh + jax.lax.dot_general(t, wrel[slh, :],
                                          (((1,), (1,)), ((), ())),
                                          preferred_element_type=jnp.float32)
            oh = oh * inv
            oh = oh + brel_ref[:, slh]
            outs[h] = oh

    out = jnp.concatenate(outs, axis=1)
    o_ref[0] = jax.lax.dot_general(out, wproj_ref[...],
                                   (((1,), (1,)), ((), ())),
                                   preferred_element_type=jnp.float32
                                   ) + bproj_ref[...]


@jax.jit
def kernel(x, rel, Wqkv, Wproj, bproj, Wrel, brel):
    nblk = _N // _BQ

    qkv = pl.pallas_call(
        _qkv_kernel,
        grid=(_B, nblk),
        in_specs=[
            pl.BlockSpec((1, _BQ, _C), lambda b, i: (b, i, 0)),
            pl.BlockSpec((3 * _C, _C), lambda b, i: (0, 0)),
        ],
        out_specs=pl.BlockSpec((1, _BQ, 3 * _C), lambda b, i: (b, i, 0)),
        out_shape=jax.ShapeDtypeStruct((_B, _N, 3 * _C), jnp.float32),
        compiler_params=pltpu.CompilerParams(
            dimension_semantics=("parallel", "parallel")),
    )(x, Wqkv)
    q = qkv[:, :, :_C]
    k = qkv[:, :, _C:2 * _C]
    v = qkv[:, :, 2 * _C:]

    centers = rel[:, 0, :, 0:3]
    ccol = jnp.concatenate(
        [centers, jnp.zeros((_B, _N, 5), jnp.float32)], axis=-1)
    crow = jnp.transpose(ccol, (0, 2, 1))
    rel_t = jnp.transpose(rel, (0, 3, 1, 2))

    out = pl.pallas_call(
        _attn_kernel,
        grid=(_B, nblk),
        in_specs=[
            pl.BlockSpec((1, _BQ, _C), lambda b, i: (b, i, 0)),
            pl.BlockSpec((1, _N, _C), lambda b, i: (b, 0, 0)),
            pl.BlockSpec((1, _N, _C), lambda b, i: (b, 0, 0)),
            pl.BlockSpec((1, 12, _BQ, _N), lambda b, i: (b, 0, i, 0)),
            pl.BlockSpec((1, _BQ, 8), lambda b, i: (b, i, 0)),
            pl.BlockSpec((1, 8, _N), lambda b, i: (b, 0, 0)),
            pl.BlockSpec((_C, 12), lambda b, i: (0, 0)),
            pl.BlockSpec((_C, _C), lambda b, i: (0, 0)),
            pl.BlockSpec((1, _C), lambda b, i: (0, 0)),
            pl.BlockSpec((1, _C), lambda b, i: (0, 0)),
        ],
        out_specs=pl.BlockSpec((1, _BQ, _C), lambda b, i: (b, i, 0)),
        out_shape=jax.ShapeDtypeStruct((_B, _N, _C), jnp.float32),
        compiler_params=pltpu.CompilerParams(
            dimension_semantics=("parallel", "parallel")),
    )(q, k, v, rel_t, ccol, crow, Wrel, Wproj,
      bproj.reshape(1, _C), brel.reshape(1, _C))
    return out
